# baseline (speedup 1.0000x reference)
import functools

import jax
import jax.numpy as jnp
from jax import lax
from jax.experimental import pallas as pl
from jax.experimental.pallas import tpu as pltpu
from jax.experimental.pallas import tpu_sc as plsc

NQ, NC, K, C, H = 4096, 16384, 16, 128, 8
AUX = 16
DT = 2 * C
TN_ATTN = 512
TN_PROJ = 512
CH = 64
NBUF = 4
NSEG = 4

SC_CORES = 2
SC_SUBCORES = 16
NW = SC_CORES * SC_SUBCORES

BF = jnp.bfloat16
F32 = jnp.float32
I32 = jnp.int32


def _table_body(cf_ref, cc_ref, w16_ref, wk_ref, bk_ref, wv_ref, bv_ref,
                out_ref):
    cf = cf_ref[...].astype(BF)
    k = jnp.dot(cf, wk_ref[...].astype(BF), preferred_element_type=F32) \
        + bk_ref[...]
    v = jnp.dot(cf, wv_ref[...].astype(BF), preferred_element_type=F32) \
        + bv_ref[...]
    kb = lax.bitcast_convert_type(k.astype(BF), jnp.uint16).astype(I32)
    vb = lax.bitcast_convert_type(v.astype(BF), jnp.uint16).astype(I32)
    cp = cc_ref[...] @ w16_ref[...]
    out_ref[:, 0:C] = (kb << 16) | vb
    out_ref[:, C:DT] = lax.bitcast_convert_type(cp, I32)


def _attn_body(g_ref, qc_ref, qf_ref, w16_ref, bp1_ref, wq_ref, bq_ref,
               wp2_ref, bp2_ref, ww1_ref, bw1_ref, ww2_ref, bw2_ref,
               wo_ref, bo_ref, out_ref):
    tn = qf_ref.shape[0]
    kt = K * tn
    g = g_ref[...]
    u = g[:, :, 0:C]
    kg = lax.bitcast_convert_type(u, F32)
    vg = lax.bitcast_convert_type(u << 16, F32)
    cpw = lax.bitcast_convert_type(g[:, :, C:DT], F32)

    qf = qf_ref[...]
    w16 = w16_ref[...]
    qp = qc_ref[...] @ w16 + bp1_ref[...]
    q = (jnp.dot(qf.astype(BF), wq_ref[...].astype(BF),
                 preferred_element_type=F32) + bq_ref[...])

    posw = qp[None, :, :] - cpw
    pw = jnp.maximum(posw, 0.0).astype(BF).reshape(kt, C)
    pe = (jnp.dot(pw, wp2_ref[...].astype(BF), preferred_element_type=F32)
          + bp2_ref[...])
    rel = (q[None, :, :] - kg).reshape(kt, C) + pe
    t = jnp.maximum(
        jnp.dot(rel.astype(BF), ww1_ref[...].astype(BF),
                preferred_element_type=F32) + bw1_ref[...], 0.0)
    w = (jnp.dot(t.astype(BF), ww2_ref[...].astype(BF),
                 preferred_element_type=F32) + bw2_ref[...])

    w3 = w.reshape(K, tn, H)
    m = jnp.max(w3, axis=0)
    e = jnp.exp(w3 - m[None])
    s = jnp.sum(e, axis=0)
    attn = (e / s[None]).reshape(kt, H)

    hc = lax.broadcasted_iota(I32, (H, C), 1) // (C // H)
    hr = lax.broadcasted_iota(I32, (H, C), 0)
    expand = (hc == hr).astype(F32)
    af = (attn @ expand).reshape(K, tn, C)

    val = vg + pe.reshape(K, tn, C)
    out = jnp.sum(af * val, axis=0)
    out_ref[...] = (qf
                    + jnp.dot(out.astype(BF), wo_ref[...].astype(BF),
                              preferred_element_type=F32) + bo_ref[...])


def _table(cf, cc16, w16, wk, bk, wv, bv):
    m = cf.shape[0]
    grid = (m // TN_PROJ,)
    full = lambda shape: pl.BlockSpec(shape, lambda i: (0, 0))
    return pl.pallas_call(
        _table_body,
        grid=grid,
        in_specs=[
            pl.BlockSpec((TN_PROJ, C), lambda i: (i, 0)),
            pl.BlockSpec((TN_PROJ, AUX), lambda i: (i, 0)),
            full((AUX, C)),
            full((C, C)), full((1, C)), full((C, C)), full((1, C)),
        ],
        out_specs=pl.BlockSpec((TN_PROJ, DT), lambda i: (i, 0)),
        out_shape=jax.ShapeDtypeStruct((m, DT), I32),
    )(cf, cc16, w16, wk, bk.reshape(1, C), wv, bv.reshape(1, C))


def _attention(g3, qc16, qf, w16, p):
    n = qf.shape[0]
    grid = (n // TN_ATTN,)
    full = lambda shape: pl.BlockSpec(shape, lambda i: (0, 0))
    return pl.pallas_call(
        _attn_body,
        grid=grid,
        in_specs=[
            pl.BlockSpec((K, TN_ATTN, DT), lambda i: (0, i, 0)),
            pl.BlockSpec((TN_ATTN, AUX), lambda i: (i, 0)),
            pl.BlockSpec((TN_ATTN, C), lambda i: (i, 0)),
            full((AUX, C)), full((1, C)),
            full((C, C)), full((1, C)),
            full((C, C)), full((1, C)),
            full((C, C)), full((1, C)),
            full((C, H)), full((1, H)),
            full((C, C)), full((1, C)),
        ],
        out_specs=pl.BlockSpec((TN_ATTN, C), lambda i: (i, 0)),
        out_shape=jax.ShapeDtypeStruct((n, C), F32),
    )(g3, qc16, qf,
      w16, p['bp1'].reshape(1, C),
      p['Wq'], p['bq'].reshape(1, C),
      p['Wp2'], p['bp2'].reshape(1, C),
      p['Ww1'], p['bw1'].reshape(1, C),
      p['Ww2'], p['bw2'].reshape(1, H),
      p['Wo'], p['bo'].reshape(1, C))


def _sc_gather(table, idx):
    b = idx.shape[0]
    per_w = b // NW
    nch = per_w // CH
    ngrp = nch // NBUF
    mesh = plsc.VectorSubcoreMesh(core_axis_name="c", subcore_axis_name="s")

    @functools.partial(
        pl.kernel,
        mesh=mesh,
        out_type=jax.ShapeDtypeStruct((b, DT), I32),
        scratch_types=(
            [pltpu.VMEM((per_w,), I32)]
            + [pltpu.VMEM((CH, DT), I32) for _ in range(NBUF)]
            + [pltpu.SemaphoreType.DMA for _ in range(2 * NBUF)]
        ),
    )
    def gk(table_hbm, idx_hbm, out_hbm, idx_v, *rest):
        bufs = rest[:NBUF]
        gsems = rest[NBUF:2 * NBUF]
        ssems = rest[2 * NBUF:]
        wid = lax.axis_index("s") * SC_CORES + lax.axis_index("c")
        base = wid * per_w
        pltpu.sync_copy(idx_hbm.at[pl.ds(base, per_w)], idx_v)

        def group(grp, carry):
            cbase = grp * (NBUF * CH)
            gcps = []
            for bi in range(NBUF):
                @pl.when(grp > 0)
                def _wait_store(bi=bi):
                    pltpu.make_async_copy(
                        bufs[bi], out_hbm.at[pl.ds(base, CH)],
                        ssems[bi]).wait()
                gcps.append(pltpu.async_copy(
                    table_hbm.at[idx_v.at[pl.ds(cbase + bi * CH, CH)]],
                    bufs[bi], gsems[bi]))
            for bi in range(NBUF):
                gcps[bi].wait()
                pltpu.async_copy(
                    bufs[bi],
                    out_hbm.at[pl.ds(base + cbase + bi * CH, CH)],
                    ssems[bi])
            return carry

        lax.fori_loop(0, ngrp, group, 0)
        for bi in range(NBUF):
            pltpu.make_async_copy(
                bufs[bi], out_hbm.at[pl.ds(base, CH)], ssems[bi]).wait()

    return gk(table, idx)


def _block(p, w16, qfeat, qc16, cfeat, cc16, knn):
    n = qfeat.shape[0]
    tbl = _table(cfeat, cc16, w16, p['Wk'], p['bk'], p['Wv'], p['bv'])
    knn_t = knn.astype(I32).T
    ns = n // NSEG
    outs = []
    for s in range(NSEG):
        idx = knn_t[:, s * ns:(s + 1) * ns].reshape(-1)
        g = _sc_gather(tbl, idx)
        g3 = g.reshape(K, ns, DT)
        outs.append(_attention(g3, qc16[s * ns:(s + 1) * ns],
                               qfeat[s * ns:(s + 1) * ns], w16, p))
    return jnp.concatenate(outs, axis=0)


def _pad_aux(x):
    return jnp.pad(x, ((0, 0), (0, AUX - x.shape[1])))


def kernel(query_coord, query_feat, query_offset, context_coord, context_feat,
           context_offset, knn_query2query, knn_query2context,
           knn_context2query, params_query_attn, params_context_attn):
    qc16 = _pad_aux(query_coord)
    cc16 = _pad_aux(context_coord)
    w16_q = jnp.pad(params_query_attn['Wp1'], ((0, AUX - 3), (0, 0)))
    w16_c = jnp.pad(params_context_attn['Wp1'], ((0, AUX - 3), (0, 0)))

    qf = _block(params_query_attn, w16_q, query_feat, qc16,
                query_feat, qc16, knn_query2query)
    qf = _block(params_context_attn, w16_c, qf, qc16,
                context_feat, cc16, knn_query2context)
    cf = _block(params_context_attn, w16_c, context_feat, cc16,
                qf, qc16, knn_context2query)
    return (query_coord, qf, query_offset, context_coord, cf, context_offset)


# speedup vs baseline: 1.0443x; 1.0443x over previous
import functools

import jax
import jax.numpy as jnp
from jax import lax
from jax.experimental import pallas as pl
from jax.experimental.pallas import tpu as pltpu
from jax.experimental.pallas import tpu_sc as plsc

NQ, NC, K, C, H = 4096, 16384, 16, 128, 8
AUX = 16
DT = 2 * C
TN_ATTN = 512
TN_PROJ = 512
CH = 64
NBUF = 4
NSEG = 4

SC_CORES = 2
SC_SUBCORES = 16
NW = SC_CORES * SC_SUBCORES

BF = jnp.bfloat16
F32 = jnp.float32
I32 = jnp.int32


def _table_body(cf_ref, cc_ref, w16_ref, wk_ref, bk_ref, wv_ref, bv_ref,
                out_ref):
    cf = cf_ref[...].astype(BF)
    k = jnp.dot(cf, wk_ref[...].astype(BF), preferred_element_type=F32) \
        + bk_ref[...]
    v = jnp.dot(cf, wv_ref[...].astype(BF), preferred_element_type=F32) \
        + bv_ref[...]
    kb = lax.bitcast_convert_type(k.astype(BF), jnp.uint16).astype(I32)
    vb = lax.bitcast_convert_type(v.astype(BF), jnp.uint16).astype(I32)
    cp = cc_ref[...] @ w16_ref[...]
    out_ref[:, 0:C] = (kb << 16) | vb
    out_ref[:, C:DT] = lax.bitcast_convert_type(cp, I32)


def _attn_body(g_ref, qc_ref, qf_ref, w16_ref, bp1_ref, wq_ref, bq_ref,
               wp2_ref, bp2_ref, ww1_ref, bw1_ref, ww2_ref, bw2_ref,
               wo_ref, bo_ref, out_ref):
    tn = qf_ref.shape[0]
    kt = K * tn
    g = g_ref[...]
    u = g[:, :, 0:C]
    kg = lax.bitcast_convert_type(u, F32)
    vg = lax.bitcast_convert_type(u << 16, F32)
    cpw = lax.bitcast_convert_type(g[:, :, C:DT], F32)

    qf = qf_ref[...]
    w16 = w16_ref[...]
    qp = qc_ref[...] @ w16 + bp1_ref[...]
    q = (jnp.dot(qf.astype(BF), wq_ref[...].astype(BF),
                 preferred_element_type=F32) + bq_ref[...])

    posw = qp[None, :, :] - cpw
    pw = jnp.maximum(posw.astype(BF), 0).reshape(kt, C)
    pe = (jnp.dot(pw, wp2_ref[...].astype(BF), preferred_element_type=F32)
          + bp2_ref[...])
    rel = (q[None, :, :] - kg).reshape(kt, C) + pe
    t = jnp.maximum(
        (jnp.dot(rel.astype(BF), ww1_ref[...].astype(BF),
                 preferred_element_type=F32) + bw1_ref[...]).astype(BF), 0)
    w = (jnp.dot(t, ww2_ref[...].astype(BF),
                 preferred_element_type=F32) + bw2_ref[...])

    w3 = w.reshape(K, tn, H)
    m = jnp.max(w3, axis=0)
    e = jnp.exp(w3 - m[None])
    s = jnp.sum(e, axis=0)
    attn = (e / s[None]).reshape(kt, H)

    hc = lax.broadcasted_iota(I32, (H, C), 1) // (C // H)
    hr = lax.broadcasted_iota(I32, (H, C), 0)
    expand = (hc == hr).astype(F32)
    af = (attn @ expand).reshape(K, tn, C)

    val = vg + pe.reshape(K, tn, C)
    out = jnp.sum(af * val, axis=0)
    out_ref[...] = (qf
                    + jnp.dot(out.astype(BF), wo_ref[...].astype(BF),
                              preferred_element_type=F32) + bo_ref[...])


def _table(cf, cc16, w16, wk, bk, wv, bv):
    m = cf.shape[0]
    grid = (m // TN_PROJ,)
    full = lambda shape: pl.BlockSpec(shape, lambda i: (0, 0))
    return pl.pallas_call(
        _table_body,
        grid=grid,
        in_specs=[
            pl.BlockSpec((TN_PROJ, C), lambda i: (i, 0)),
            pl.BlockSpec((TN_PROJ, AUX), lambda i: (i, 0)),
            full((AUX, C)),
            full((C, C)), full((1, C)), full((C, C)), full((1, C)),
        ],
        out_specs=pl.BlockSpec((TN_PROJ, DT), lambda i: (i, 0)),
        out_shape=jax.ShapeDtypeStruct((m, DT), I32),
    )(cf, cc16, w16, wk, bk.reshape(1, C), wv, bv.reshape(1, C))


def _attention(g3, qc16, qf, w16, p):
    n = qf.shape[0]
    grid = (n // TN_ATTN,)
    full = lambda shape: pl.BlockSpec(shape, lambda i: (0, 0))
    return pl.pallas_call(
        _attn_body,
        grid=grid,
        in_specs=[
            pl.BlockSpec((K, TN_ATTN, DT), lambda i: (0, i, 0)),
            pl.BlockSpec((TN_ATTN, AUX), lambda i: (i, 0)),
            pl.BlockSpec((TN_ATTN, C), lambda i: (i, 0)),
            full((AUX, C)), full((1, C)),
            full((C, C)), full((1, C)),
            full((C, C)), full((1, C)),
            full((C, C)), full((1, C)),
            full((C, H)), full((1, H)),
            full((C, C)), full((1, C)),
        ],
        out_specs=pl.BlockSpec((TN_ATTN, C), lambda i: (i, 0)),
        out_shape=jax.ShapeDtypeStruct((n, C), F32),
    )(g3, qc16, qf,
      w16, p['bp1'].reshape(1, C),
      p['Wq'], p['bq'].reshape(1, C),
      p['Wp2'], p['bp2'].reshape(1, C),
      p['Ww1'], p['bw1'].reshape(1, C),
      p['Ww2'], p['bw2'].reshape(1, H),
      p['Wo'], p['bo'].reshape(1, C))


def _sc_gather(table, idx):
    b = idx.shape[0]
    per_w = b // NW
    nch = per_w // CH
    ngrp = nch // NBUF
    mesh = plsc.VectorSubcoreMesh(core_axis_name="c", subcore_axis_name="s")

    @functools.partial(
        pl.kernel,
        mesh=mesh,
        out_type=jax.ShapeDtypeStruct((b, DT), I32),
        scratch_types=(
            [pltpu.VMEM((per_w,), I32)]
            + [pltpu.VMEM((CH, DT), I32) for _ in range(NBUF)]
            + [pltpu.SemaphoreType.DMA for _ in range(2 * NBUF)]
        ),
    )
    def gk(table_hbm, idx_hbm, out_hbm, idx_v, *rest):
        bufs = rest[:NBUF]
        gsems = rest[NBUF:2 * NBUF]
        ssems = rest[2 * NBUF:]
        wid = lax.axis_index("s") * SC_CORES + lax.axis_index("c")
        base = wid * per_w
        pltpu.sync_copy(idx_hbm.at[pl.ds(base, per_w)], idx_v)

        def group(grp, carry):
            cbase = grp * (NBUF * CH)
            gcps = []
            for bi in range(NBUF):
                @pl.when(grp > 0)
                def _wait_store(bi=bi):
                    pltpu.make_async_copy(
                        bufs[bi], out_hbm.at[pl.ds(base, CH)],
                        ssems[bi]).wait()
                gcps.append(pltpu.async_copy(
                    table_hbm.at[idx_v.at[pl.ds(cbase + bi * CH, CH)]],
                    bufs[bi], gsems[bi]))
            for bi in range(NBUF):
                gcps[bi].wait()
                pltpu.async_copy(
                    bufs[bi],
                    out_hbm.at[pl.ds(base + cbase + bi * CH, CH)],
                    ssems[bi])
            return carry

        lax.fori_loop(0, ngrp, group, 0)
        for bi in range(NBUF):
            pltpu.make_async_copy(
                bufs[bi], out_hbm.at[pl.ds(base, CH)], ssems[bi]).wait()

    return gk(table, idx)


def _block(p, w16, qfeat, qc16, cfeat, cc16, knn):
    n = qfeat.shape[0]
    tbl = _table(cfeat, cc16, w16, p['Wk'], p['bk'], p['Wv'], p['bv'])
    idx = knn.astype(I32).T.reshape(-1)
    g = _sc_gather(tbl, idx)
    g3 = g.reshape(K, n, DT)
    return _attention(g3, qc16, qfeat, w16, p)


def _pad_aux(x):
    return jnp.pad(x, ((0, 0), (0, AUX - x.shape[1])))


def kernel(query_coord, query_feat, query_offset, context_coord, context_feat,
           context_offset, knn_query2query, knn_query2context,
           knn_context2query, params_query_attn, params_context_attn):
    qc16 = _pad_aux(query_coord)
    cc16 = _pad_aux(context_coord)
    w16_q = jnp.pad(params_query_attn['Wp1'], ((0, AUX - 3), (0, 0)))
    w16_c = jnp.pad(params_context_attn['Wp1'], ((0, AUX - 3), (0, 0)))

    qf = _block(params_query_attn, w16_q, query_feat, qc16,
                query_feat, qc16, knn_query2query)
    qf = _block(params_context_attn, w16_c, qf, qc16,
                context_feat, cc16, knn_query2context)
    cf = _block(params_context_attn, w16_c, context_feat, cc16,
                qf, qc16, knn_context2query)
    return (query_coord, qf, query_offset, context_coord, cf, context_offset)


# speedup vs baseline: 1.1359x; 1.0877x over previous
import functools

import jax
import jax.numpy as jnp
from jax import lax
from jax.experimental import pallas as pl
from jax.experimental.pallas import tpu as pltpu
from jax.experimental.pallas import tpu_sc as plsc

NQ, NC, K, C, H = 4096, 16384, 16, 128, 8
AUX = 16
TN_ATTN = 512
TN_PROJ = 512
CH = 128
NBUF = 4
LANES = 16
CPAD = 16

SC_CORES = 2
SC_SUBCORES = 16
NW = SC_CORES * SC_SUBCORES

BF = jnp.bfloat16
F32 = jnp.float32
I32 = jnp.int32


def _table_body(cf_ref, wk_ref, bk_ref, wv_ref, bv_ref, out_ref):
    cf = cf_ref[...].astype(BF)
    k = jnp.dot(cf, wk_ref[...].astype(BF), preferred_element_type=F32) \
        + bk_ref[...]
    v = jnp.dot(cf, wv_ref[...].astype(BF), preferred_element_type=F32) \
        + bv_ref[...]
    kb = lax.bitcast_convert_type(k.astype(BF), jnp.uint16).astype(I32)
    vb = lax.bitcast_convert_type(v.astype(BF), jnp.uint16).astype(I32)
    out_ref[...] = (kb << 16) | vb


def _attn_body(g_ref, aux_ref, qc_ref, qf_ref, w16_ref, bp1_ref, wq_ref,
               bq_ref, wp2_ref, bp2_ref, ww1_ref, bw1_ref, ww2_ref, bw2_ref,
               wo_ref, bo_ref, out_ref):
    tn = qf_ref.shape[0]
    kt = K * tn
    u = g_ref[...]
    kg = lax.bitcast_convert_type(u, F32)
    vg = lax.bitcast_convert_type(u << 16, F32)

    qf = qf_ref[...]
    w16 = w16_ref[...]
    qp = qc_ref[...] @ w16 + bp1_ref[...]
    q = (jnp.dot(qf.astype(BF), wq_ref[...].astype(BF),
                 preferred_element_type=F32) + bq_ref[...])

    auxp = aux_ref[...]
    auxt = jnp.transpose(auxp, (0, 1, 3, 2)).reshape(kt, 3)
    cpw = (auxt @ w16_ref[0:3, :]).reshape(K, tn, C)

    posw = qp[None, :, :] - cpw
    pw = jnp.maximum(posw.astype(BF), 0).reshape(kt, C)
    pe = (jnp.dot(pw, wp2_ref[...].astype(BF), preferred_element_type=F32)
          + bp2_ref[...])
    rel = (q[None, :, :] - kg).reshape(kt, C) + pe
    t = jnp.maximum(
        (jnp.dot(rel.astype(BF), ww1_ref[...].astype(BF),
                 preferred_element_type=F32) + bw1_ref[...]).astype(BF), 0)
    w = (jnp.dot(t, ww2_ref[...].astype(BF),
                 preferred_element_type=F32) + bw2_ref[...])

    w3 = w.reshape(K, tn, H)
    m = jnp.max(w3, axis=0)
    e = jnp.exp(w3 - m[None])
    s = jnp.sum(e, axis=0)
    attn = (e / s[None]).reshape(kt, H)

    hc = lax.broadcasted_iota(I32, (H, C), 1) // (C // H)
    hr = lax.broadcasted_iota(I32, (H, C), 0)
    expand = (hc == hr).astype(F32)
    af = (attn @ expand).reshape(K, tn, C)

    val = vg + pe.reshape(K, tn, C)
    out = jnp.sum(af * val, axis=0)
    out_ref[...] = (qf
                    + jnp.dot(out.astype(BF), wo_ref[...].astype(BF),
                              preferred_element_type=F32) + bo_ref[...])


def _table(cf, wk, bk, wv, bv):
    m = cf.shape[0]
    grid = (m // TN_PROJ,)
    full = lambda shape: pl.BlockSpec(shape, lambda i: (0, 0))
    return pl.pallas_call(
        _table_body,
        grid=grid,
        in_specs=[
            pl.BlockSpec((TN_PROJ, C), lambda i: (i, 0)),
            full((C, C)), full((1, C)), full((C, C)), full((1, C)),
        ],
        out_specs=pl.BlockSpec((TN_PROJ, C), lambda i: (i, 0)),
        out_shape=jax.ShapeDtypeStruct((m, C), I32),
    )(cf, wk, bk.reshape(1, C), wv, bv.reshape(1, C))


def _attention(g3, aux4, qc16, qf, w16, p):
    n = qf.shape[0]
    grid = (n // TN_ATTN,)
    full = lambda shape: pl.BlockSpec(shape, lambda i: (0, 0))
    return pl.pallas_call(
        _attn_body,
        grid=grid,
        in_specs=[
            pl.BlockSpec((K, TN_ATTN, C), lambda i: (0, i, 0)),
            pl.BlockSpec((K, TN_ATTN // CH, 3, CH), lambda i: (0, i, 0, 0)),
            pl.BlockSpec((TN_ATTN, AUX), lambda i: (i, 0)),
            pl.BlockSpec((TN_ATTN, C), lambda i: (i, 0)),
            full((AUX, C)), full((1, C)),
            full((C, C)), full((1, C)),
            full((C, C)), full((1, C)),
            full((C, C)), full((1, C)),
            full((C, H)), full((1, H)),
            full((C, C)), full((1, C)),
        ],
        out_specs=pl.BlockSpec((TN_ATTN, C), lambda i: (i, 0)),
        out_shape=jax.ShapeDtypeStruct((n, C), F32),
    )(g3, aux4, qc16, qf,
      w16, p['bp1'].reshape(1, C),
      p['Wq'], p['bq'].reshape(1, C),
      p['Wp2'], p['bp2'].reshape(1, C),
      p['Ww1'], p['bw1'].reshape(1, C),
      p['Ww2'], p['bw2'].reshape(1, H),
      p['Wo'], p['bo'].reshape(1, C))


def _sc_gather(table, coords_flat, idx):
    b = idx.shape[0]
    per_w = b // NW
    nch = per_w // CH
    ngrp = nch // NBUF
    mesh = plsc.VectorSubcoreMesh(core_axis_name="c", subcore_axis_name="s")

    @functools.partial(
        pl.kernel,
        mesh=mesh,
        out_type=[jax.ShapeDtypeStruct((b, C), I32),
                  jax.ShapeDtypeStruct((b * 3,), F32)],
        scratch_types=(
            [pltpu.VMEM((per_w,), I32)]
            + [pltpu.VMEM((CH, C), I32) for _ in range(NBUF)]
            + [pltpu.VMEM((3 * CH,), I32) for _ in range(NBUF)]
            + [pltpu.VMEM((3 * CH,), F32) for _ in range(NBUF)]
            + [pltpu.SemaphoreType.DMA for _ in range(4 * NBUF)]
        ),
    )
    def gk(table_hbm, coords_hbm, idx_hbm, kv_hbm, cc_hbm, idx_v, *rest):
        kvb = rest[:NBUF]
        posb = rest[NBUF:2 * NBUF]
        cb = rest[2 * NBUF:3 * NBUF]
        gsems = rest[3 * NBUF:4 * NBUF]
        csems = rest[4 * NBUF:5 * NBUF]
        s1sems = rest[5 * NBUF:6 * NBUF]
        s2sems = rest[6 * NBUF:7 * NBUF]
        wid = lax.axis_index("s") * SC_CORES + lax.axis_index("c")
        base = wid * per_w
        pltpu.sync_copy(idx_hbm.at[pl.ds(base, per_w)], idx_v)

        def group(grp, carry):
            cbase = grp * (NBUF * CH)
            kvcps, ccps = [], []
            for bi in range(NBUF):
                @pl.when(grp > 0)
                def _drain(bi=bi):
                    pltpu.make_async_copy(
                        kvb[bi], kv_hbm.at[pl.ds(base, CH)],
                        s1sems[bi]).wait()
                    pltpu.make_async_copy(
                        cb[bi], cc_hbm.at[pl.ds(0, 3 * CH)],
                        s2sems[bi]).wait()
                coff = cbase + bi * CH
                kvcps.append(pltpu.async_copy(
                    table_hbm.at[idx_v.at[pl.ds(coff, CH)]],
                    kvb[bi], gsems[bi]))
                for j in range(CH // LANES):
                    iv = idx_v[pl.ds(coff + j * LANES, LANES)]
                    p16 = iv * CPAD
                    for c3 in range(3):
                        posb[bi][pl.ds(c3 * CH + j * LANES, LANES)] = p16 + c3
                ccps.append(pltpu.async_copy(
                    coords_hbm.at[posb[bi]], cb[bi], csems[bi]))
            for bi in range(NBUF):
                coff = cbase + bi * CH
                kvcps[bi].wait()
                ccps[bi].wait()
                pltpu.async_copy(
                    kvb[bi], kv_hbm.at[pl.ds(base + coff, CH)], s1sems[bi])
                pltpu.async_copy(
                    cb[bi], cc_hbm.at[pl.ds((base + coff) * 3, 3 * CH)],
                    s2sems[bi])
            return carry

        lax.fori_loop(0, ngrp, group, 0)
        for bi in range(NBUF):
            pltpu.make_async_copy(
                kvb[bi], kv_hbm.at[pl.ds(base, CH)], s1sems[bi]).wait()
            pltpu.make_async_copy(
                cb[bi], cc_hbm.at[pl.ds(0, 3 * CH)], s2sems[bi]).wait()

    return gk(table, coords_flat, idx)


def _block(p, w16, qfeat, qc16, cfeat, ccflat, knn):
    n = qfeat.shape[0]
    tbl = _table(cfeat, p['Wk'], p['bk'], p['Wv'], p['bv'])
    idx = knn.astype(I32).T.reshape(-1)
    kv, cc = _sc_gather(tbl, ccflat, idx)
    g3 = kv.reshape(K, n, C)
    aux4 = cc.reshape(K, n // CH, 3, CH)
    return _attention(g3, aux4, qc16, qfeat, w16, p)


def _pad_aux(x):
    return jnp.pad(x, ((0, 0), (0, AUX - x.shape[1])))


def kernel(query_coord, query_feat, query_offset, context_coord, context_feat,
           context_offset, knn_query2query, knn_query2context,
           knn_context2query, params_query_attn, params_context_attn):
    qc16 = _pad_aux(query_coord)
    qcflat = qc16.reshape(-1)
    ccflat = _pad_aux(context_coord).reshape(-1)
    w16_q = jnp.pad(params_query_attn['Wp1'], ((0, AUX - 3), (0, 0)))
    w16_c = jnp.pad(params_context_attn['Wp1'], ((0, AUX - 3), (0, 0)))

    qf = _block(params_query_attn, w16_q, query_feat, qc16,
                query_feat, qcflat, knn_query2query)
    qf = _block(params_context_attn, w16_c, qf, qc16,
                context_feat, ccflat, knn_query2context)
    cf = _block(params_context_attn, w16_c, context_feat,
                _pad_aux(context_coord), qf, qcflat, knn_context2query)
    return (query_coord, qf, query_offset, context_coord, cf, context_offset)


# speedup vs baseline: 1.1513x; 1.0136x over previous
import functools

import jax
import jax.numpy as jnp
from jax import lax
from jax.experimental import pallas as pl
from jax.experimental.pallas import tpu as pltpu
from jax.experimental.pallas import tpu_sc as plsc

NQ, NC, K, C, H = 4096, 16384, 16, 128, 8
AUX = 16
TN_ATTN = 1024
TN_PROJ = 512
CH = 128
NBUF = 4
LANES = 16
CPAD = 16

SC_CORES = 2
SC_SUBCORES = 16
NW = SC_CORES * SC_SUBCORES

BF = jnp.bfloat16
F32 = jnp.float32
I32 = jnp.int32


def _table_body(cf_ref, wk_ref, bk_ref, wv_ref, bv_ref, out_ref):
    cf = cf_ref[...].astype(BF)
    k = jnp.dot(cf, wk_ref[...].astype(BF), preferred_element_type=F32) \
        + bk_ref[...]
    v = jnp.dot(cf, wv_ref[...].astype(BF), preferred_element_type=F32) \
        + bv_ref[...]
    kb = lax.bitcast_convert_type(k.astype(BF), jnp.uint16).astype(I32)
    vb = lax.bitcast_convert_type(v.astype(BF), jnp.uint16).astype(I32)
    out_ref[...] = (kb << 16) | vb


def _attn_body(g_ref, aux_ref, qc_ref, qf_ref, w16_ref, bp1_ref, wq_ref,
               bq_ref, wp2_ref, bp2_ref, ww1_ref, bw1_ref, ww2_ref, bw2_ref,
               wo_ref, bo_ref, out_ref):
    tn = qf_ref.shape[0]
    kt = K * tn
    u = g_ref[...]
    kg = lax.bitcast_convert_type(u, F32)
    vg = lax.bitcast_convert_type(u << 16, F32)

    qf = qf_ref[...]
    w16 = w16_ref[...]
    qp = qc_ref[...] @ w16 + bp1_ref[...]
    q = (jnp.dot(qf.astype(BF), wq_ref[...].astype(BF),
                 preferred_element_type=F32) + bq_ref[...])

    auxp = aux_ref[...]
    auxt = jnp.transpose(auxp, (0, 1, 3, 2)).reshape(kt, 3)
    cpw = (auxt @ w16_ref[0:3, :]).reshape(K, tn, C)

    posw = qp[None, :, :] - cpw
    pw = jnp.maximum(posw.astype(BF), 0).reshape(kt, C)
    pe = (jnp.dot(pw, wp2_ref[...].astype(BF), preferred_element_type=F32)
          + bp2_ref[...])
    rel = (q[None, :, :] - kg).reshape(kt, C) + pe
    t = jnp.maximum(
        (jnp.dot(rel.astype(BF), ww1_ref[...].astype(BF),
                 preferred_element_type=F32) + bw1_ref[...]).astype(BF), 0)
    w = (jnp.dot(t, ww2_ref[...].astype(BF),
                 preferred_element_type=F32) + bw2_ref[...])

    w3 = w.reshape(K, tn, H)
    m = jnp.max(w3, axis=0)
    e = jnp.exp(w3 - m[None])
    s = jnp.sum(e, axis=0)
    attn = (e / s[None]).reshape(kt, H)

    hc = lax.broadcasted_iota(I32, (H, C), 1) // (C // H)
    hr = lax.broadcasted_iota(I32, (H, C), 0)
    expand = (hc == hr).astype(F32)
    af = (attn @ expand).reshape(K, tn, C)

    val = vg + pe.reshape(K, tn, C)
    out = jnp.sum(af * val, axis=0)
    out_ref[...] = (qf
                    + jnp.dot(out.astype(BF), wo_ref[...].astype(BF),
                              preferred_element_type=F32) + bo_ref[...])


def _table(cf, wk, bk, wv, bv):
    m = cf.shape[0]
    grid = (m // TN_PROJ,)
    full = lambda shape: pl.BlockSpec(shape, lambda i: (0, 0))
    return pl.pallas_call(
        _table_body,
        grid=grid,
        in_specs=[
            pl.BlockSpec((TN_PROJ, C), lambda i: (i, 0)),
            full((C, C)), full((1, C)), full((C, C)), full((1, C)),
        ],
        out_specs=pl.BlockSpec((TN_PROJ, C), lambda i: (i, 0)),
        out_shape=jax.ShapeDtypeStruct((m, C), I32),
    )(cf, wk, bk.reshape(1, C), wv, bv.reshape(1, C))


def _attention(g3, aux4, qc16, qf, w16, p):
    n = qf.shape[0]
    grid = (n // TN_ATTN,)
    full = lambda shape: pl.BlockSpec(shape, lambda i: (0, 0))
    return pl.pallas_call(
        _attn_body,
        grid=grid,
        in_specs=[
            pl.BlockSpec((K, TN_ATTN, C), lambda i: (0, i, 0)),
            pl.BlockSpec((K, TN_ATTN // CH, 3, CH), lambda i: (0, i, 0, 0)),
            pl.BlockSpec((TN_ATTN, AUX), lambda i: (i, 0)),
            pl.BlockSpec((TN_ATTN, C), lambda i: (i, 0)),
            full((AUX, C)), full((1, C)),
            full((C, C)), full((1, C)),
            full((C, C)), full((1, C)),
            full((C, C)), full((1, C)),
            full((C, H)), full((1, H)),
            full((C, C)), full((1, C)),
        ],
        out_specs=pl.BlockSpec((TN_ATTN, C), lambda i: (i, 0)),
        out_shape=jax.ShapeDtypeStruct((n, C), F32),
    )(g3, aux4, qc16, qf,
      w16, p['bp1'].reshape(1, C),
      p['Wq'], p['bq'].reshape(1, C),
      p['Wp2'], p['bp2'].reshape(1, C),
      p['Ww1'], p['bw1'].reshape(1, C),
      p['Ww2'], p['bw2'].reshape(1, H),
      p['Wo'], p['bo'].reshape(1, C))


def _sc_gather(table, coords_flat, idx):
    b = idx.shape[0]
    per_w = b // NW
    nch = per_w // CH
    ngrp = nch // NBUF
    mesh = plsc.VectorSubcoreMesh(core_axis_name="c", subcore_axis_name="s")

    @functools.partial(
        pl.kernel,
        mesh=mesh,
        out_type=[jax.ShapeDtypeStruct((b, C), I32),
                  jax.ShapeDtypeStruct((b * 3,), F32)],
        scratch_types=(
            [pltpu.VMEM((per_w,), I32)]
            + [pltpu.VMEM((CH, C), I32) for _ in range(NBUF)]
            + [pltpu.VMEM((3 * CH,), I32) for _ in range(NBUF)]
            + [pltpu.VMEM((3 * CH,), F32) for _ in range(NBUF)]
            + [pltpu.SemaphoreType.DMA for _ in range(4 * NBUF)]
        ),
    )
    def gk(table_hbm, coords_hbm, idx_hbm, kv_hbm, cc_hbm, idx_v, *rest):
        kvb = rest[:NBUF]
        posb = rest[NBUF:2 * NBUF]
        cb = rest[2 * NBUF:3 * NBUF]
        gsems = rest[3 * NBUF:4 * NBUF]
        csems = rest[4 * NBUF:5 * NBUF]
        s1sems = rest[5 * NBUF:6 * NBUF]
        s2sems = rest[6 * NBUF:7 * NBUF]
        wid = lax.axis_index("s") * SC_CORES + lax.axis_index("c")
        base = wid * per_w
        pltpu.sync_copy(idx_hbm.at[pl.ds(base, per_w)], idx_v)

        def group(grp, carry):
            cbase = grp * (NBUF * CH)
            kvcps, ccps = [], []
            for bi in range(NBUF):
                @pl.when(grp > 0)
                def _drain(bi=bi):
                    pltpu.make_async_copy(
                        kvb[bi], kv_hbm.at[pl.ds(base, CH)],
                        s1sems[bi]).wait()
                    pltpu.make_async_copy(
                        cb[bi], cc_hbm.at[pl.ds(0, 3 * CH)],
                        s2sems[bi]).wait()
                coff = cbase + bi * CH
                kvcps.append(pltpu.async_copy(
                    table_hbm.at[idx_v.at[pl.ds(coff, CH)]],
                    kvb[bi], gsems[bi]))
                for j in range(CH // LANES):
                    iv = idx_v[pl.ds(coff + j * LANES, LANES)]
                    p16 = iv * CPAD
                    for c3 in range(3):
                        posb[bi][pl.ds(c3 * CH + j * LANES, LANES)] = p16 + c3
                ccps.append(pltpu.async_copy(
                    coords_hbm.at[posb[bi]], cb[bi], csems[bi]))
            for bi in range(NBUF):
                coff = cbase + bi * CH
                kvcps[bi].wait()
                ccps[bi].wait()
                pltpu.async_copy(
                    kvb[bi], kv_hbm.at[pl.ds(base + coff, CH)], s1sems[bi])
                pltpu.async_copy(
                    cb[bi], cc_hbm.at[pl.ds((base + coff) * 3, 3 * CH)],
                    s2sems[bi])
            return carry

        lax.fori_loop(0, ngrp, group, 0)
        for bi in range(NBUF):
            pltpu.make_async_copy(
                kvb[bi], kv_hbm.at[pl.ds(base, CH)], s1sems[bi]).wait()
            pltpu.make_async_copy(
                cb[bi], cc_hbm.at[pl.ds(0, 3 * CH)], s2sems[bi]).wait()

    return gk(table, coords_flat, idx)


def _block(p, w16, qfeat, qc16, cfeat, ccflat, knn):
    n = qfeat.shape[0]
    tbl = _table(cfeat, p['Wk'], p['bk'], p['Wv'], p['bv'])
    idx = knn.astype(I32).T.reshape(-1)
    kv, cc = _sc_gather(tbl, ccflat, idx)
    g3 = kv.reshape(K, n, C)
    aux4 = cc.reshape(K, n // CH, 3, CH)
    return _attention(g3, aux4, qc16, qfeat, w16, p)


def _pad_aux(x):
    return jnp.pad(x, ((0, 0), (0, AUX - x.shape[1])))


def kernel(query_coord, query_feat, query_offset, context_coord, context_feat,
           context_offset, knn_query2query, knn_query2context,
           knn_context2query, params_query_attn, params_context_attn):
    qc16 = _pad_aux(query_coord)
    qcflat = qc16.reshape(-1)
    ccflat = _pad_aux(context_coord).reshape(-1)
    w16_q = jnp.pad(params_query_attn['Wp1'], ((0, AUX - 3), (0, 0)))
    w16_c = jnp.pad(params_context_attn['Wp1'], ((0, AUX - 3), (0, 0)))

    qf = _block(params_query_attn, w16_q, query_feat, qc16,
                query_feat, qcflat, knn_query2query)
    qf = _block(params_context_attn, w16_c, qf, qc16,
                context_feat, ccflat, knn_query2context)
    cf = _block(params_context_attn, w16_c, context_feat,
                _pad_aux(context_coord), qf, qcflat, knn_context2query)
    return (query_coord, qf, query_offset, context_coord, cf, context_offset)
